# baseline (speedup 1.0000x reference)
import functools

import jax
import jax.numpy as jnp
from jax.experimental import pallas as pl
from jax.experimental.pallas import tpu as pltpu
from jax.experimental.pallas import tpu_sc as plsc

_EPS = 1e-5
_GATHER_WIN = 128
_NSEG = 5
_P = 2000


def _sc_gather(comb, idx2d, e_pad):
    n_col = comb.shape[1]
    mesh = plsc.VectorSubcoreMesh(core_axis_name="core",
                                  subcore_axis_name="subcore")

    @functools.partial(
        pl.kernel,
        out_type=jax.ShapeDtypeStruct((e_pad, n_col), jnp.float32),
        mesh=mesh,
        compiler_params=pltpu.CompilerParams(use_tc_tiling_on_sc=True),
    )
    def sc_kernel(comb_hbm, idx_hbm, out_hbm):
        def body(idx_vmem, out_vmem):
            pltpu.sync_copy(comb_hbm.at[idx_vmem.at[0]], out_vmem)

        pltpu.emit_pipeline(
            body,
            grid=(e_pad // _GATHER_WIN,),
            in_specs=[
                pl.BlockSpec((1, _GATHER_WIN), lambda i: (0, i)),
            ],
            out_specs=[
                pl.BlockSpec((_GATHER_WIN, n_col), lambda i: (i, 0)),
            ],
            core_axis_name=("core", "subcore"),
            dimension_semantics=(pltpu.PARALLEL,),
        )(idx_hbm, out_hbm)

    return sc_kernel(comb, idx2d)


def _weight_net(loc_t, w1_ref, b1_ref, w2_ref, b2_ref, w3_ref, b3_ref):
    h = b1_ref[...]
    for d in range(3):
        h = h + w1_ref[:, d:d + 1] * loc_t[d:d + 1, :]
    h = jnp.maximum(h, 0.0)
    h2 = b2_ref[...]
    for d in range(8):
        h2 = h2 + w2_ref[:, d:d + 1] * h[d:d + 1, :]
    h2 = jnp.maximum(h2, 0.0)
    wv = b3_ref[...]
    for d in range(8):
        wv = wv + w3_ref[:, d:d + 1] * h2[d:d + 1, :]
    return jnp.maximum(wv, 0.0)


def _tc_body(n_blocks, n_pair,
             gca_ref, gcb_ref, xyz_ref, ft_ref,
             w1_ref, b1_ref, w2_ref, b2_ref, w3_ref, b3_ref,
             lw_ref, lb_ref, u2w_ref, u2b_ref, scw_ref, scb_ref,
             out_ref, loc_ref, m_ref, ls_ref):
    t = pl.program_id(1)
    xyzv = xyz_ref[...]

    halves = []
    loc_ts = []
    for half, gc_ref in enumerate((gca_ref, gcb_ref)):
        loc_t = (gc_ref[:, 64:67] - xyzv).T
        loc_ts.append(loc_t)
        wv = _weight_net(loc_t, w1_ref, b1_ref, w2_ref, b2_ref,
                         w3_ref, b3_ref)
        gf_t = gc_ref[:, 0:64].T
        halves.append((wv, gf_t))
    ls_ref[pl.ds(8 * t, 6), :] = jnp.concatenate(loc_ts, axis=0)

    n_w = halves[0][0].shape[0]
    for j in range(n_w):
        piece = (jnp.broadcast_to(halves[0][0][j:j + 1, :],
                                  halves[0][1].shape) * halves[0][1]
                 + jnp.broadcast_to(halves[1][0][j:j + 1, :],
                                    halves[1][1].shape) * halves[1][1])
        sl = slice(j * 64, (j + 1) * 64)

        @pl.when(t == 0)
        def _(piece=piece, sl=sl):
            m_ref[sl, :] = piece

        @pl.when(t > 0)
        def _(piece=piece, sl=sl):
            m_ref[sl, :] = m_ref[sl, :] + piece

    @pl.when(t == n_pair - 1)
    def _():
        m = m_ref[...]
        o = jnp.dot(lw_ref[...], m, preferred_element_type=jnp.float32)
        o = jnp.maximum(o + lb_ref[...], 0.0)
        o = jnp.dot(u2w_ref[...], o, preferred_element_type=jnp.float32)
        o = o + u2b_ref[...]
        s = jnp.dot(scw_ref[...], ft_ref[...].T,
                    preferred_element_type=jnp.float32)
        s = s + scb_ref[...]
        tt = o + s
        tt = jnp.where(tt >= 0.0, tt, 0.1 * tt)
        out_ref[...] = tt.T
        ls_t = ls_ref[...].T
        loc_ref[...] = jnp.concatenate(
            [ls_t[:, 8 * q:8 * q + 6] for q in range(n_pair)], axis=1)


def kernel(dense_xyz, dense_feats, nei_inds,
           wn_w1, wn_b1, wn_g1, wn_be1,
           wn_w2, wn_b2, wn_g2, wn_be2,
           wn_w3, wn_b3, wn_g3, wn_be3,
           lin_w, lin_b,
           u2_w, u2_b, u2_g, u2_be,
           sc_w, sc_b, sc_g, sc_be):
    B, N, _ = dense_xyz.shape
    K = nei_inds.shape[2]
    Cin = dense_feats.shape[2]
    Cout = u2_w.shape[1]

    xyz = dense_xyz[0]
    feats = dense_feats[0]
    comb = jnp.concatenate(
        [feats, xyz, jnp.zeros((N, 61), jnp.float32)], axis=1)

    inv = 1.0 / jnp.sqrt(1.0 + _EPS)
    s1 = wn_g1 * inv
    w1t = (wn_w1 * s1[None, :]).T
    b1c = (wn_b1 * s1 + wn_be1)[:, None]
    s2 = wn_g2 * inv
    w2t = (wn_w2 * s2[None, :]).T
    b2c = (wn_b2 * s2 + wn_be2)[:, None]
    s3 = wn_g3 * inv
    w3t = (wn_w3 * s3[None, :]).T
    b3c = (wn_b3 * s3 + wn_be3)[:, None]
    wn_out = w3t.shape[0]
    linp = lin_w.reshape(Cin, wn_out, -1).transpose(2, 1, 0).reshape(
        -1, Cin * wn_out)
    lb = lin_b[:, None]
    su2 = u2_g * inv
    u2wf = (u2_w * su2[None, :]).T
    u2bf = (u2_b * su2 + u2_be)[:, None]
    ssc = sc_g * inv
    scwf = (sc_w * ssc[None, :]).T
    scbf = (sc_b * ssc + sc_be)[:, None]

    nc = N // _NSEG
    n_blocks = nc // _P
    chunk = 32 * _GATHER_WIN
    e_seg = nc * K
    e_pad = ((e_seg + chunk - 1) // chunk) * chunk

    kma = lambda pi, t: (2 * t * n_blocks + pi, 0)
    kmb = lambda pi, t: ((2 * t + 1) * n_blocks + pi, 0)
    fixed = lambda pi, t: (0, 0)

    idx_t = jnp.transpose(nei_inds[0], (1, 0)).astype(jnp.int32)
    gathered = []
    for s in range(_NSEG):
        idx_km = idx_t[:, s * nc:(s + 1) * nc].reshape(-1)
        idx2d = jnp.pad(idx_km, (0, e_pad - e_seg)).reshape(1, e_pad)
        gathered.append(_sc_gather(comb, idx2d, e_pad))

    outs, locs = [], []
    for s in range(_NSEG):
        gc_all = gathered[s]
        out_s, loc_s = pl.pallas_call(
            functools.partial(_tc_body, n_blocks, K // 2),
            grid=(n_blocks, K // 2),
            in_specs=[
                pl.BlockSpec((_P, comb.shape[1]), kma),
                pl.BlockSpec((_P, comb.shape[1]), kmb),
                pl.BlockSpec((_P, 3), lambda pi, t: (pi, 0)),
                pl.BlockSpec((_P, Cin), lambda pi, t: (pi, 0)),
                pl.BlockSpec(w1t.shape, fixed),
                pl.BlockSpec(b1c.shape, fixed),
                pl.BlockSpec(w2t.shape, fixed),
                pl.BlockSpec(b2c.shape, fixed),
                pl.BlockSpec(w3t.shape, fixed),
                pl.BlockSpec(b3c.shape, fixed),
                pl.BlockSpec(linp.shape, fixed),
                pl.BlockSpec(lb.shape, fixed),
                pl.BlockSpec(u2wf.shape, fixed),
                pl.BlockSpec(u2bf.shape, fixed),
                pl.BlockSpec(scwf.shape, fixed),
                pl.BlockSpec(scbf.shape, fixed),
            ],
            out_specs=[
                pl.BlockSpec((_P, Cout), lambda pi, t: (pi, 0)),
                pl.BlockSpec((_P, 3 * K), lambda pi, t: (pi, 0)),
            ],
            out_shape=[
                jax.ShapeDtypeStruct((nc, Cout), jnp.float32),
                jax.ShapeDtypeStruct((nc, 3 * K), jnp.float32),
            ],
            scratch_shapes=[pltpu.VMEM((Cin * wn_out, _P), jnp.float32),
                            pltpu.VMEM((4 * K, _P), jnp.float32)],
            compiler_params=pltpu.CompilerParams(
                dimension_semantics=("parallel", "arbitrary")),
        )(gc_all, gc_all, xyz[s * nc:(s + 1) * nc], feats[s * nc:(s + 1) * nc],
          w1t, b1c, w2t, b2c, w3t, b3c,
          linp, lb, u2wf, u2bf, scwf, scbf)
        outs.append(out_s)
        locs.append(loc_s)

    new_feat = jnp.concatenate(outs, axis=0)[None]
    weight_net_input = jnp.concatenate(locs, axis=0).reshape(
        1, N, K, 3)
    return new_feat, weight_net_input


# speedup vs baseline: 1.3519x; 1.3519x over previous
import functools

import jax
import jax.numpy as jnp
from jax.experimental import pallas as pl
from jax.experimental.pallas import tpu as pltpu
from jax.experimental.pallas import tpu_sc as plsc

_EPS = 1e-5
_GATHER_WIN = 128
_NSEG = 2
_P = 1000


def _sc_gather(comb, idx2d, e_pad):
    n_col = comb.shape[1]
    mesh = plsc.VectorSubcoreMesh(core_axis_name="core",
                                  subcore_axis_name="subcore")

    @functools.partial(
        pl.kernel,
        out_type=jax.ShapeDtypeStruct((e_pad, n_col), jnp.float32),
        mesh=mesh,
        compiler_params=pltpu.CompilerParams(use_tc_tiling_on_sc=True),
    )
    def sc_kernel(comb_hbm, idx_hbm, out_hbm):
        def body(idx_vmem, out_vmem):
            pltpu.sync_copy(comb_hbm.at[idx_vmem.at[0]], out_vmem)

        pltpu.emit_pipeline(
            body,
            grid=(e_pad // _GATHER_WIN,),
            in_specs=[
                pl.BlockSpec((1, _GATHER_WIN), lambda i: (0, i)),
            ],
            out_specs=[
                pl.BlockSpec((_GATHER_WIN, n_col), lambda i: (i, 0)),
            ],
            core_axis_name=("core", "subcore"),
            dimension_semantics=(pltpu.PARALLEL,),
        )(idx_hbm, out_hbm)

    return sc_kernel(comb, idx2d)


def _weight_net(loc_t, w1_ref, b1_ref, w2_ref, b2_ref, w3_ref, b3_ref):
    h = b1_ref[...]
    for d in range(3):
        h = h + w1_ref[:, d:d + 1] * loc_t[d:d + 1, :]
    h = jnp.maximum(h, 0.0)
    h2 = b2_ref[...]
    for d in range(8):
        h2 = h2 + w2_ref[:, d:d + 1] * h[d:d + 1, :]
    h2 = jnp.maximum(h2, 0.0)
    wv = b3_ref[...]
    for d in range(8):
        wv = wv + w3_ref[:, d:d + 1] * h2[d:d + 1, :]
    return jnp.maximum(wv, 0.0)


def _tc_body(n_blocks, n_pair,
             gca_ref, gcb_ref, xyz_ref, ft_ref,
             w1_ref, b1_ref, w2_ref, b2_ref, w3_ref, b3_ref,
             lw_ref, lb_ref, u2w_ref, u2b_ref, scw_ref, scb_ref,
             out_ref, loc_ref, m_ref, ls_ref):
    t = pl.program_id(1)
    xyzv = xyz_ref[...]

    halves = []
    loc_ts = []
    for half, gc_ref in enumerate((gca_ref, gcb_ref)):
        loc_t = (gc_ref[:, 64:67] - xyzv).T
        loc_ts.append(loc_t)
        wv = _weight_net(loc_t, w1_ref, b1_ref, w2_ref, b2_ref,
                         w3_ref, b3_ref)
        gf_t = gc_ref[:, 0:64].T
        halves.append((wv, gf_t))
    ls_ref[pl.ds(8 * t, 6), :] = jnp.concatenate(loc_ts, axis=0)

    n_w = halves[0][0].shape[0]
    for j in range(n_w):
        piece = (jnp.broadcast_to(halves[0][0][j:j + 1, :],
                                  halves[0][1].shape) * halves[0][1]
                 + jnp.broadcast_to(halves[1][0][j:j + 1, :],
                                    halves[1][1].shape) * halves[1][1])
        sl = slice(j * 64, (j + 1) * 64)

        @pl.when(t == 0)
        def _(piece=piece, sl=sl):
            m_ref[sl, :] = piece

        @pl.when(t > 0)
        def _(piece=piece, sl=sl):
            m_ref[sl, :] = m_ref[sl, :] + piece

    @pl.when(t == n_pair - 1)
    def _():
        m = m_ref[...]
        o = jnp.dot(lw_ref[...], m, preferred_element_type=jnp.float32)
        o = jnp.maximum(o + lb_ref[...], 0.0)
        o = jnp.dot(u2w_ref[...], o, preferred_element_type=jnp.float32)
        o = o + u2b_ref[...]
        s = jnp.dot(scw_ref[...], ft_ref[...].T,
                    preferred_element_type=jnp.float32)
        s = s + scb_ref[...]
        tt = o + s
        tt = jnp.where(tt >= 0.0, tt, 0.1 * tt)
        out_ref[...] = tt.T
        ls_t = ls_ref[...].T
        loc_ref[...] = jnp.concatenate(
            [ls_t[:, 8 * q:8 * q + 6] for q in range(n_pair)], axis=1)


def kernel(dense_xyz, dense_feats, nei_inds,
           wn_w1, wn_b1, wn_g1, wn_be1,
           wn_w2, wn_b2, wn_g2, wn_be2,
           wn_w3, wn_b3, wn_g3, wn_be3,
           lin_w, lin_b,
           u2_w, u2_b, u2_g, u2_be,
           sc_w, sc_b, sc_g, sc_be):
    B, N, _ = dense_xyz.shape
    K = nei_inds.shape[2]
    Cin = dense_feats.shape[2]
    Cout = u2_w.shape[1]

    xyz = dense_xyz[0]
    feats = dense_feats[0]
    comb = jnp.concatenate(
        [feats, xyz, jnp.zeros((N, 61), jnp.float32)], axis=1)

    inv = 1.0 / jnp.sqrt(1.0 + _EPS)
    s1 = wn_g1 * inv
    w1t = (wn_w1 * s1[None, :]).T
    b1c = (wn_b1 * s1 + wn_be1)[:, None]
    s2 = wn_g2 * inv
    w2t = (wn_w2 * s2[None, :]).T
    b2c = (wn_b2 * s2 + wn_be2)[:, None]
    s3 = wn_g3 * inv
    w3t = (wn_w3 * s3[None, :]).T
    b3c = (wn_b3 * s3 + wn_be3)[:, None]
    wn_out = w3t.shape[0]
    linp = lin_w.reshape(Cin, wn_out, -1).transpose(2, 1, 0).reshape(
        -1, Cin * wn_out)
    lb = lin_b[:, None]
    su2 = u2_g * inv
    u2wf = (u2_w * su2[None, :]).T
    u2bf = (u2_b * su2 + u2_be)[:, None]
    ssc = sc_g * inv
    scwf = (sc_w * ssc[None, :]).T
    scbf = (sc_b * ssc + sc_be)[:, None]

    nc = N // _NSEG
    n_blocks = nc // _P
    chunk = 32 * _GATHER_WIN
    e_seg = nc * K
    e_pad = ((e_seg + chunk - 1) // chunk) * chunk

    kma = lambda pi, t: (2 * t * n_blocks + pi, 0)
    kmb = lambda pi, t: ((2 * t + 1) * n_blocks + pi, 0)
    fixed = lambda pi, t: (0, 0)

    idx_t = jnp.transpose(nei_inds[0], (1, 0)).astype(jnp.int32)
    gathered = []
    for s in range(_NSEG):
        idx_km = idx_t[:, s * nc:(s + 1) * nc].reshape(-1)
        idx2d = jnp.pad(idx_km, (0, e_pad - e_seg)).reshape(1, e_pad)
        gathered.append(_sc_gather(comb, idx2d, e_pad))

    outs, locs = [], []
    for s in range(_NSEG):
        gc_all = gathered[s]
        out_s, loc_s = pl.pallas_call(
            functools.partial(_tc_body, n_blocks, K // 2),
            grid=(n_blocks, K // 2),
            in_specs=[
                pl.BlockSpec((_P, comb.shape[1]), kma),
                pl.BlockSpec((_P, comb.shape[1]), kmb),
                pl.BlockSpec((_P, 3), lambda pi, t: (pi, 0)),
                pl.BlockSpec((_P, Cin), lambda pi, t: (pi, 0)),
                pl.BlockSpec(w1t.shape, fixed),
                pl.BlockSpec(b1c.shape, fixed),
                pl.BlockSpec(w2t.shape, fixed),
                pl.BlockSpec(b2c.shape, fixed),
                pl.BlockSpec(w3t.shape, fixed),
                pl.BlockSpec(b3c.shape, fixed),
                pl.BlockSpec(linp.shape, fixed),
                pl.BlockSpec(lb.shape, fixed),
                pl.BlockSpec(u2wf.shape, fixed),
                pl.BlockSpec(u2bf.shape, fixed),
                pl.BlockSpec(scwf.shape, fixed),
                pl.BlockSpec(scbf.shape, fixed),
            ],
            out_specs=[
                pl.BlockSpec((_P, Cout), lambda pi, t: (pi, 0)),
                pl.BlockSpec((_P, 3 * K), lambda pi, t: (pi, 0)),
            ],
            out_shape=[
                jax.ShapeDtypeStruct((nc, Cout), jnp.float32),
                jax.ShapeDtypeStruct((nc, 3 * K), jnp.float32),
            ],
            scratch_shapes=[pltpu.VMEM((Cin * wn_out, _P), jnp.float32),
                            pltpu.VMEM((4 * K, _P), jnp.float32)],
            compiler_params=pltpu.CompilerParams(
                dimension_semantics=("parallel", "arbitrary")),
        )(gc_all, gc_all, xyz[s * nc:(s + 1) * nc], feats[s * nc:(s + 1) * nc],
          w1t, b1c, w2t, b2c, w3t, b3c,
          linp, lb, u2wf, u2bf, scwf, scbf)
        outs.append(out_s)
        locs.append(loc_s)

    new_feat = jnp.concatenate(outs, axis=0)[None]
    weight_net_input = jnp.concatenate(locs, axis=0).reshape(
        1, N, K, 3)
    return new_feat, weight_net_input


# speedup vs baseline: 1.8116x; 1.3400x over previous
import functools

import jax
import jax.numpy as jnp
from jax.experimental import pallas as pl
from jax.experimental.pallas import tpu as pltpu
from jax.experimental.pallas import tpu_sc as plsc

_EPS = 1e-5
_GATHER_WIN = 128
_NSEG = 2
_P = 1000


def _sc_gather(comb, idx2d, e_pad):
    n_col = comb.shape[1]
    mesh = plsc.VectorSubcoreMesh(core_axis_name="core",
                                  subcore_axis_name="subcore")

    @functools.partial(
        pl.kernel,
        out_type=jax.ShapeDtypeStruct((e_pad, n_col), jnp.float32),
        mesh=mesh,
        compiler_params=pltpu.CompilerParams(use_tc_tiling_on_sc=True),
    )
    def sc_kernel(comb_hbm, idx_hbm, out_hbm):
        def body(idx_vmem, out_vmem):
            pltpu.sync_copy(comb_hbm.at[idx_vmem.at[0]], out_vmem)

        pltpu.emit_pipeline(
            body,
            grid=(e_pad // _GATHER_WIN,),
            in_specs=[
                pl.BlockSpec((1, _GATHER_WIN), lambda i: (0, i)),
            ],
            out_specs=[
                pl.BlockSpec((_GATHER_WIN, n_col), lambda i: (i, 0)),
            ],
            core_axis_name=("core", "subcore"),
            dimension_semantics=(pltpu.PARALLEL,),
        )(idx_hbm, out_hbm)

    return sc_kernel(comb, idx2d)


def _weight_net(loc_t, w1_ref, b1_ref, w2_ref, b2_ref, w3_ref, b3_ref):
    h = b1_ref[...]
    for d in range(3):
        h = h + w1_ref[:, d:d + 1] * loc_t[d:d + 1, :]
    h = jnp.maximum(h, 0.0)
    h2 = b2_ref[...]
    for d in range(8):
        h2 = h2 + w2_ref[:, d:d + 1] * h[d:d + 1, :]
    h2 = jnp.maximum(h2, 0.0)
    wv = b3_ref[...]
    for d in range(8):
        wv = wv + w3_ref[:, d:d + 1] * h2[d:d + 1, :]
    return jnp.maximum(wv, 0.0)


def _tc_body(n_k, *refs):
    gc_refs = refs[:n_k]
    (xyz_ref, ft_ref,
     w1_ref, b1_ref, w2_ref, b2_ref, w3_ref, b3_ref,
     lw_ref, lb_ref, u2w_ref, u2b_ref, scw_ref, scb_ref,
     out_ref, loc_ref, m_ref, gft_ref, wv_ref) = refs[n_k:]

    xyzv = xyz_ref[...]
    loc_ts = []
    for k in range(n_k):
        gc = gc_refs[k]
        loc_t = (gc[:, 64:67] - xyzv).T
        loc_ts.append(loc_t)
        wv_ref[16 * k:16 * (k + 1), :] = _weight_net(
            loc_t, w1_ref, b1_ref, w2_ref, b2_ref, w3_ref, b3_ref)
        gft_ref[64 * k:64 * (k + 1), :] = gc[:, 0:64].T

    n_w = wv_ref.shape[0] // n_k
    for j in range(n_w):
        acc = None
        for k in range(n_k):
            b = jnp.broadcast_to(wv_ref[16 * k + j:16 * k + j + 1, :],
                                 (64, wv_ref.shape[1]))
            g = gft_ref[64 * k:64 * (k + 1), :]
            acc = b * g if acc is None else acc + b * g
        m_ref[64 * j:64 * (j + 1), :] = acc

    m = m_ref[...]
    o = jnp.dot(lw_ref[...], m, preferred_element_type=jnp.float32)
    o = jnp.maximum(o + lb_ref[...], 0.0)
    o = jnp.dot(u2w_ref[...], o, preferred_element_type=jnp.float32)
    o = o + u2b_ref[...]
    s = jnp.dot(scw_ref[...], ft_ref[...].T,
                preferred_element_type=jnp.float32)
    s = s + scb_ref[...]
    tt = o + s
    tt = jnp.where(tt >= 0.0, tt, 0.1 * tt)
    out_ref[...] = tt.T
    loc_ref[...] = jnp.concatenate(loc_ts, axis=0).T


def kernel(dense_xyz, dense_feats, nei_inds,
           wn_w1, wn_b1, wn_g1, wn_be1,
           wn_w2, wn_b2, wn_g2, wn_be2,
           wn_w3, wn_b3, wn_g3, wn_be3,
           lin_w, lin_b,
           u2_w, u2_b, u2_g, u2_be,
           sc_w, sc_b, sc_g, sc_be):
    B, N, _ = dense_xyz.shape
    K = nei_inds.shape[2]
    Cin = dense_feats.shape[2]
    Cout = u2_w.shape[1]

    xyz = dense_xyz[0]
    feats = dense_feats[0]
    comb = jnp.concatenate(
        [feats, xyz, jnp.zeros((N, 61), jnp.float32)], axis=1)

    inv = 1.0 / jnp.sqrt(1.0 + _EPS)
    s1 = wn_g1 * inv
    w1t = (wn_w1 * s1[None, :]).T
    b1c = (wn_b1 * s1 + wn_be1)[:, None]
    s2 = wn_g2 * inv
    w2t = (wn_w2 * s2[None, :]).T
    b2c = (wn_b2 * s2 + wn_be2)[:, None]
    s3 = wn_g3 * inv
    w3t = (wn_w3 * s3[None, :]).T
    b3c = (wn_b3 * s3 + wn_be3)[:, None]
    wn_out = w3t.shape[0]
    linp = lin_w.reshape(Cin, wn_out, -1).transpose(2, 1, 0).reshape(
        -1, Cin * wn_out)
    lb = lin_b[:, None]
    su2 = u2_g * inv
    u2wf = (u2_w * su2[None, :]).T
    u2bf = (u2_b * su2 + u2_be)[:, None]
    ssc = sc_g * inv
    scwf = (sc_w * ssc[None, :]).T
    scbf = (sc_b * ssc + sc_be)[:, None]

    nc = N // _NSEG
    n_blocks = nc // _P
    chunk = 32 * _GATHER_WIN
    e_seg = nc * K
    e_pad = ((e_seg + chunk - 1) // chunk) * chunk

    fixed = lambda pi: (0, 0)

    idx_t = jnp.transpose(nei_inds[0], (1, 0)).astype(jnp.int32)
    gathered = []
    for s in range(_NSEG):
        idx_km = idx_t[:, s * nc:(s + 1) * nc].reshape(-1)
        idx2d = jnp.pad(idx_km, (0, e_pad - e_seg)).reshape(1, e_pad)
        gathered.append(_sc_gather(comb, idx2d, e_pad))

    outs, locs = [], []
    for s in range(_NSEG):
        gc_all = gathered[s]
        gc_specs = [
            pl.BlockSpec((_P, comb.shape[1]),
                         (lambda pi, kk=k: (kk * n_blocks + pi, 0)))
            for k in range(K)
        ]
        out_s, loc_s = pl.pallas_call(
            functools.partial(_tc_body, K),
            grid=(n_blocks,),
            in_specs=gc_specs + [
                pl.BlockSpec((_P, 3), lambda pi: (pi, 0)),
                pl.BlockSpec((_P, Cin), lambda pi: (pi, 0)),
                pl.BlockSpec(w1t.shape, fixed),
                pl.BlockSpec(b1c.shape, fixed),
                pl.BlockSpec(w2t.shape, fixed),
                pl.BlockSpec(b2c.shape, fixed),
                pl.BlockSpec(w3t.shape, fixed),
                pl.BlockSpec(b3c.shape, fixed),
                pl.BlockSpec(linp.shape, fixed),
                pl.BlockSpec(lb.shape, fixed),
                pl.BlockSpec(u2wf.shape, fixed),
                pl.BlockSpec(u2bf.shape, fixed),
                pl.BlockSpec(scwf.shape, fixed),
                pl.BlockSpec(scbf.shape, fixed),
            ],
            out_specs=[
                pl.BlockSpec((_P, Cout), lambda pi: (pi, 0)),
                pl.BlockSpec((_P, 3 * K), lambda pi: (pi, 0)),
            ],
            out_shape=[
                jax.ShapeDtypeStruct((nc, Cout), jnp.float32),
                jax.ShapeDtypeStruct((nc, 3 * K), jnp.float32),
            ],
            scratch_shapes=[pltpu.VMEM((Cin * wn_out, _P), jnp.float32),
                            pltpu.VMEM((Cin * K, _P), jnp.float32),
                            pltpu.VMEM((wn_out * K, _P), jnp.float32)],
            compiler_params=pltpu.CompilerParams(
                dimension_semantics=("parallel",)),
        )(*([gc_all] * K),
          xyz[s * nc:(s + 1) * nc], feats[s * nc:(s + 1) * nc],
          w1t, b1c, w2t, b2c, w3t, b3c,
          linp, lb, u2wf, u2bf, scwf, scbf)
        outs.append(out_s)
        locs.append(loc_s)

    new_feat = jnp.concatenate(outs, axis=0)[None]
    weight_net_input = jnp.concatenate(locs, axis=0).reshape(
        1, N, K, 3)
    return new_feat, weight_net_input
